# baseline (speedup 1.0000x reference)
import functools

import jax
import jax.numpy as jnp
from jax import lax
from jax.experimental import pallas as pl
from jax.experimental.pallas import tpu as pltpu
from jax.experimental.pallas import tpu_sc as plsc

N = 10000
E = 320000
D = 128

NC = 2
NS = 16
NW = NC * NS
L = 16
CPT = D // NW

BLK = 2000
CHUNK = 4000
NCHUNK = E // CHUNK
GROUPS = CHUNK // L
BG = 5
SLAB = N * CPT


def _prep_body(x_r, pos_r, combos_r, w1a_r, w1b_r, w1c_r, b1_r, a_r, p_r, t_r):
    pos_b = pos_r[...]
    pw = (pos_b[:, 0:1] * w1b_r[0:1, :]
          + pos_b[:, 1:2] * w1b_r[1:2, :]
          + pos_b[:, 2:3] * w1b_r[2:3, :])
    p_r[...] = pw
    a_r[...] = (jnp.dot(x_r[...], w1a_r[...],
                        preferred_element_type=jnp.float32)
                + pw + b1_r[...])
    cb = combos_r[...]
    t_r[...] = (cb[:, 0:1] * w1c_r[0:1, :]
                + cb[:, 1:2] * w1c_r[1:2, :]
                + cb[:, 2:3] * w1c_r[2:3, :])


_prep_call = pl.pallas_call(
    _prep_body,
    grid=(N // BLK,),
    in_specs=[
        pl.BlockSpec((BLK, D), lambda i: (i, 0)),
        pl.BlockSpec((BLK, 3), lambda i: (i, 0)),
        pl.BlockSpec((32, 3), lambda i: (0, 0)),
        pl.BlockSpec((D, D), lambda i: (0, 0)),
        pl.BlockSpec((3, D), lambda i: (0, 0)),
        pl.BlockSpec((3, D), lambda i: (0, 0)),
        pl.BlockSpec((1, D), lambda i: (0, 0)),
    ],
    out_specs=[
        pl.BlockSpec((BLK, D), lambda i: (i, 0)),
        pl.BlockSpec((BLK, D), lambda i: (i, 0)),
        pl.BlockSpec((32, D), lambda i: (0, 0)),
    ],
    out_shape=[
        jax.ShapeDtypeStruct((N, D), jnp.float32),
        jax.ShapeDtypeStruct((N, D), jnp.float32),
        jax.ShapeDtypeStruct((32, D), jnp.float32),
    ],
)


_sc_mesh = plsc.VectorSubcoreMesh(core_axis_name="c", subcore_axis_name="s")


@functools.partial(
    pl.kernel,
    out_type=jax.ShapeDtypeStruct((NW, SLAB), jnp.float32),
    mesh=_sc_mesh,
    scratch_types=[
        pltpu.VMEM((SLAB,), jnp.float32),
        pltpu.VMEM((SLAB,), jnp.float32),
        pltpu.VMEM((32 * CPT,), jnp.float32),
        pltpu.VMEM((CHUNK,), jnp.int32),
        pltpu.VMEM((CHUNK,), jnp.int32),
    ],
    compiler_params=pltpu.CompilerParams(needs_layout_passes=False),
)
def _sc_edge_kernel(a_hbm, t_hbm, pw_hbm, q_hbm, p_hbm, out_hbm,
                    a_v, m_v, t_v, pw_v, q_v):
    wid = lax.axis_index("s") * NC + lax.axis_index("c")
    pltpu.sync_copy(a_hbm.at[wid], a_v)
    pltpu.sync_copy(t_hbm.at[wid], t_v)

    neg = jnp.full((L,), -jnp.inf, dtype=jnp.float32)

    def init_body(i, carry):
        m_v[pl.ds(i * L, L)] = neg
        return carry

    lax.fori_loop(0, SLAB // L, init_body, 0)

    def load_group(g):
        w = pw_v[pl.ds(g * L, L)]
        q = q_v[pl.ds(g * L, L)]
        p4 = (w & 0x3FFF) << 2
        c4 = (w >> 14) << 2
        q4 = q << 2
        vals = []
        for c in range(CPT):
            a_c = plsc.load_gather(a_v, [p4 + c])
            t_c = plsc.load_gather(t_v, [c4 + c])
            vals.append(a_c + t_c)
        return q4, vals

    def batch_body(b, carry):
        base = b * BG
        viol = jnp.zeros((L,), dtype=jnp.bool_)
        for j in range(BG):
            q4, vals = load_group(base + j)
            needs = []
            for c in range(CPT):
                cur = plsc.load_gather(m_v, [q4 + c])
                needs.append(vals[c] > cur)
            for c in range(CPT):
                plsc.store_scatter(m_v, [q4 + c], vals[c], mask=needs[c])
            for c in range(CPT):
                cur2 = plsc.load_gather(m_v, [q4 + c])
                viol = viol | (needs[c] & (vals[c] > cur2))

        @pl.when(jnp.any(viol))
        def _slow():
            for j in range(BG):
                q4, vals = load_group(base + j)
                needs0 = []
                for c in range(CPT):
                    cur0 = plsc.load_gather(m_v, [q4 + c])
                    needs0.append(vals[c] > cur0)

                def cond(needs):
                    return jnp.any(needs[0] | needs[1] | needs[2] | needs[3])

                def body(needs):
                    for c in range(CPT):
                        plsc.store_scatter(m_v, [q4 + c], vals[c],
                                           mask=needs[c])
                    new = []
                    for c in range(CPT):
                        cur = plsc.load_gather(m_v, [q4 + c])
                        new.append(needs[c] & (vals[c] > cur))
                    return tuple(new)

                lax.while_loop(cond, body, tuple(needs0))
        return carry

    def chunk_body(k, carry):
        pltpu.sync_copy(pw_hbm.at[pl.ds(k * CHUNK, CHUNK)], pw_v)
        pltpu.sync_copy(q_hbm.at[pl.ds(k * CHUNK, CHUNK)], q_v)
        lax.fori_loop(0, GROUPS // BG, batch_body, 0)
        return carry

    lax.fori_loop(0, NCHUNK, chunk_body, 0)

    pltpu.sync_copy(p_hbm.at[wid], a_v)

    def ep_body(i, carry):
        sl = pl.ds(i * L, L)
        m_v[sl] = jnp.maximum(m_v[sl] - a_v[sl], 0.0)
        return carry

    lax.fori_loop(0, SLAB // L, ep_body, 0)
    pltpu.sync_copy(m_v, out_hbm.at[wid])


import numpy as _np

_COMBOS = _np.array(
    [[i - 1, j - 1, k - 1]
     for i in range(3) for j in range(3) for k in range(3)] + [[0, 0, 0]] * 5,
    dtype=_np.float32)


def kernel(x, pos, fps_pos, batch, frac_pos, trans_vec, scale,
           q_inds, p_inds, offset, W1, b1):
    W1a = W1[:D]
    W1b = W1[D:]
    rowsum = (trans_vec[0] * scale[0]).sum(axis=1)
    W1c = rowsum[:, None] * W1b

    A, P, T = _prep_call(x, pos, jnp.asarray(_COMBOS), W1a, W1b, W1c,
                         b1.reshape(1, D))

    A_s = A.reshape(N, NW, CPT).transpose(1, 0, 2).reshape(NW, SLAB)
    P_s = P.reshape(N, NW, CPT).transpose(1, 0, 2).reshape(NW, SLAB)
    T_s = T.reshape(32, NW, CPT).transpose(1, 0, 2).reshape(NW, 32 * CPT)

    offi = offset.astype(jnp.int32) + 1
    code = offi[:, 0] * 9 + offi[:, 1] * 3 + offi[:, 2]
    pw = p_inds.astype(jnp.int32) | (code << 14)
    qi = q_inds.astype(jnp.int32)

    out_s = _sc_edge_kernel(A_s, T_s, pw, qi, P_s)
    out = out_s.reshape(NW, N, CPT).transpose(1, 0, 2).reshape(N, D)
    return (out, pos, fps_pos, batch, frac_pos, trans_vec, scale)


# speedup vs baseline: 1.1631x; 1.1631x over previous
import functools

import jax
import jax.numpy as jnp
from jax import lax
from jax.experimental import pallas as pl
from jax.experimental.pallas import tpu as pltpu
from jax.experimental.pallas import tpu_sc as plsc

N = 10000
E = 320000
D = 128

NC = 2
NS = 16
NW = NC * NS
L = 16
CPT = D // NW

BLK = 2000
CHUNK = 8000
NCHUNK = E // CHUNK
GROUPS = CHUNK // L
BG = 10
SLAB = N * CPT


def _prep_body(x_r, pos_r, combos_r, w1a_r, w1b_r, w1c_r, b1_r, a_r, p_r, t_r):
    pos_b = pos_r[...]
    pw = (pos_b[:, 0:1] * w1b_r[0:1, :]
          + pos_b[:, 1:2] * w1b_r[1:2, :]
          + pos_b[:, 2:3] * w1b_r[2:3, :])
    p_r[...] = pw
    a_r[...] = (jnp.dot(x_r[...], w1a_r[...],
                        preferred_element_type=jnp.float32)
                + pw + b1_r[...])
    cb = combos_r[...]
    t_r[...] = (cb[:, 0:1] * w1c_r[0:1, :]
                + cb[:, 1:2] * w1c_r[1:2, :]
                + cb[:, 2:3] * w1c_r[2:3, :])


_prep_call = pl.pallas_call(
    _prep_body,
    grid=(N // BLK,),
    in_specs=[
        pl.BlockSpec((BLK, D), lambda i: (i, 0)),
        pl.BlockSpec((BLK, 3), lambda i: (i, 0)),
        pl.BlockSpec((32, 3), lambda i: (0, 0)),
        pl.BlockSpec((D, D), lambda i: (0, 0)),
        pl.BlockSpec((3, D), lambda i: (0, 0)),
        pl.BlockSpec((3, D), lambda i: (0, 0)),
        pl.BlockSpec((1, D), lambda i: (0, 0)),
    ],
    out_specs=[
        pl.BlockSpec((BLK, D), lambda i: (i, 0)),
        pl.BlockSpec((BLK, D), lambda i: (i, 0)),
        pl.BlockSpec((32, D), lambda i: (0, 0)),
    ],
    out_shape=[
        jax.ShapeDtypeStruct((N, D), jnp.float32),
        jax.ShapeDtypeStruct((N, D), jnp.float32),
        jax.ShapeDtypeStruct((32, D), jnp.float32),
    ],
)


_sc_mesh = plsc.VectorSubcoreMesh(core_axis_name="c", subcore_axis_name="s")


@functools.partial(
    pl.kernel,
    out_type=jax.ShapeDtypeStruct((NW, SLAB), jnp.float32),
    mesh=_sc_mesh,
    scratch_types=[
        pltpu.VMEM((SLAB,), jnp.float32),
        pltpu.VMEM((SLAB,), jnp.float32),
        pltpu.VMEM((32 * CPT,), jnp.float32),
        pltpu.VMEM((CHUNK,), jnp.int32),
        pltpu.VMEM((CHUNK,), jnp.int32),
        pltpu.VMEM((CHUNK,), jnp.int32),
        pltpu.VMEM((CHUNK,), jnp.int32),
        pltpu.SemaphoreType.DMA,
        pltpu.SemaphoreType.DMA,
        pltpu.SemaphoreType.DMA,
        pltpu.SemaphoreType.DMA,
    ],
    compiler_params=pltpu.CompilerParams(needs_layout_passes=False),
)
def _sc_edge_kernel(a_hbm, t_hbm, pw_hbm, q_hbm, p_hbm, out_hbm,
                    a_v, m_v, t_v, pw0, q0, pw1, q1, sp0, sq0, sp1, sq1):
    wid = lax.axis_index("s") * NC + lax.axis_index("c")
    pltpu.sync_copy(a_hbm.at[wid], a_v)
    pltpu.sync_copy(t_hbm.at[wid], t_v)

    neg = jnp.full((L,), -jnp.inf, dtype=jnp.float32)

    def init_body(i, carry):
        m_v[pl.ds(i * L, L)] = neg
        return carry

    lax.fori_loop(0, SLAB // L, init_body, 0)

    def load_group(pw_v, q_v, g):
        w = pw_v[pl.ds(g * L, L)]
        q = q_v[pl.ds(g * L, L)]
        p4 = (w & 0x3FFF) << 2
        c4 = (w >> 14) << 2
        q4 = q << 2
        vals = []
        for c in range(CPT):
            a_c = plsc.load_gather(a_v, [p4 + c])
            t_c = plsc.load_gather(t_v, [c4 + c])
            vals.append(a_c + t_c)
        return q4, vals

    def make_batch_body(pw_v, q_v):
        return functools.partial(batch_body, pw_v, q_v)

    def batch_body(pw_v, q_v, b, carry):
        base = b * BG
        viol = jnp.zeros((L,), dtype=jnp.bool_)
        for j in range(BG):
            q4, vals = load_group(pw_v, q_v, base + j)
            needs = []
            for c in range(CPT):
                cur = plsc.load_gather(m_v, [q4 + c])
                needs.append(vals[c] > cur)
            for c in range(CPT):
                plsc.store_scatter(m_v, [q4 + c], vals[c], mask=needs[c])
            for c in range(CPT):
                cur2 = plsc.load_gather(m_v, [q4 + c])
                viol = viol | (needs[c] & (vals[c] > cur2))

        @pl.when(jnp.any(viol))
        def _slow():
            for j in range(BG):
                q4, vals = load_group(pw_v, q_v, base + j)
                needs0 = []
                for c in range(CPT):
                    cur0 = plsc.load_gather(m_v, [q4 + c])
                    needs0.append(vals[c] > cur0)

                def cond(needs):
                    return jnp.any(needs[0] | needs[1] | needs[2] | needs[3])

                def body(needs):
                    for c in range(CPT):
                        plsc.store_scatter(m_v, [q4 + c], vals[c],
                                           mask=needs[c])
                    new = []
                    for c in range(CPT):
                        cur = plsc.load_gather(m_v, [q4 + c])
                        new.append(needs[c] & (vals[c] > cur))
                    return tuple(new)

                lax.while_loop(cond, body, tuple(needs0))
        return carry

    def start_copy(k, pwb, qb, sem_p, sem_q):
        kc = jnp.minimum(k, NCHUNK - 1) * CHUNK
        pltpu.async_copy(pw_hbm.at[pl.ds(kc, CHUNK)], pwb, sem_p)
        pltpu.async_copy(q_hbm.at[pl.ds(kc, CHUNK)], qb, sem_q)

    def wait_copy(pwb, qb, sem_p, sem_q):
        pltpu.make_async_copy(pw_hbm.at[pl.ds(0, CHUNK)], pwb, sem_p).wait()
        pltpu.make_async_copy(q_hbm.at[pl.ds(0, CHUNK)], qb, sem_q).wait()

    def process(pw_v, q_v):
        lax.fori_loop(0, GROUPS // BG, make_batch_body(pw_v, q_v), 0)

    start_copy(0, pw0, q0, sp0, sq0)

    def chunk_pair(kk, carry):
        k0 = kk * 2
        wait_copy(pw0, q0, sp0, sq0)
        start_copy(k0 + 1, pw1, q1, sp1, sq1)
        process(pw0, q0)
        wait_copy(pw1, q1, sp1, sq1)
        start_copy(k0 + 2, pw0, q0, sp0, sq0)
        process(pw1, q1)
        return carry

    lax.fori_loop(0, NCHUNK // 2, chunk_pair, 0)
    wait_copy(pw0, q0, sp0, sq0)

    pltpu.sync_copy(p_hbm.at[wid], a_v)

    def ep_body(i, carry):
        sl = pl.ds(i * L, L)
        m_v[sl] = jnp.maximum(m_v[sl] - a_v[sl], 0.0)
        return carry

    lax.fori_loop(0, SLAB // L, ep_body, 0)
    pltpu.sync_copy(m_v, out_hbm.at[wid])


import numpy as _np

_COMBOS = _np.array(
    [[i - 1, j - 1, k - 1]
     for i in range(3) for j in range(3) for k in range(3)] + [[0, 0, 0]] * 5,
    dtype=_np.float32)


def kernel(x, pos, fps_pos, batch, frac_pos, trans_vec, scale,
           q_inds, p_inds, offset, W1, b1):
    W1a = W1[:D]
    W1b = W1[D:]
    rowsum = (trans_vec[0] * scale[0]).sum(axis=1)
    W1c = rowsum[:, None] * W1b

    A, P, T = _prep_call(x, pos, jnp.asarray(_COMBOS), W1a, W1b, W1c,
                         b1.reshape(1, D))

    A_s = A.reshape(N, NW, CPT).transpose(1, 0, 2).reshape(NW, SLAB)
    P_s = P.reshape(N, NW, CPT).transpose(1, 0, 2).reshape(NW, SLAB)
    T_s = T.reshape(32, NW, CPT).transpose(1, 0, 2).reshape(NW, 32 * CPT)

    offi = offset.astype(jnp.int32) + 1
    code = offi[:, 0] * 9 + offi[:, 1] * 3 + offi[:, 2]
    pw = p_inds.astype(jnp.int32) | (code << 14)
    qi = q_inds.astype(jnp.int32)

    out_s = _sc_edge_kernel(A_s, T_s, pw, qi, P_s)
    out = out_s.reshape(NW, N, CPT).transpose(1, 0, 2).reshape(N, D)
    return (out, pos, fps_pos, batch, frac_pos, trans_vec, scale)


# speedup vs baseline: 1.4952x; 1.2855x over previous
import functools

import jax
import jax.numpy as jnp
from jax import lax
from jax.experimental import pallas as pl
from jax.experimental.pallas import tpu as pltpu
from jax.experimental.pallas import tpu_sc as plsc

N = 10000
E = 320000
D = 128

NC = 2
NS = 16
NW = NC * NS
L = 16
CPT = D // NW

BLK = 2000
CHUNK = 8000
NCHUNK = E // CHUNK
GROUPS = CHUNK // L
BG = 10
SLAB = N * CPT


def _prep_body(x_r, pos_r, combos_r, w1a_r, w1b_r, w1c_r, b1_r, a_r, p_r, t_r):
    pos_b = pos_r[...]
    pw = (pos_b[:, 0:1] * w1b_r[0:1, :]
          + pos_b[:, 1:2] * w1b_r[1:2, :]
          + pos_b[:, 2:3] * w1b_r[2:3, :])
    p_r[...] = pw
    a_r[...] = (jnp.dot(x_r[...], w1a_r[...],
                        preferred_element_type=jnp.float32)
                + pw + b1_r[...])
    cb = combos_r[...]
    t_r[...] = (cb[:, 0:1] * w1c_r[0:1, :]
                + cb[:, 1:2] * w1c_r[1:2, :]
                + cb[:, 2:3] * w1c_r[2:3, :])


_prep_call = pl.pallas_call(
    _prep_body,
    grid=(N // BLK,),
    in_specs=[
        pl.BlockSpec((BLK, D), lambda i: (i, 0)),
        pl.BlockSpec((BLK, 3), lambda i: (i, 0)),
        pl.BlockSpec((32, 3), lambda i: (0, 0)),
        pl.BlockSpec((D, D), lambda i: (0, 0)),
        pl.BlockSpec((3, D), lambda i: (0, 0)),
        pl.BlockSpec((3, D), lambda i: (0, 0)),
        pl.BlockSpec((1, D), lambda i: (0, 0)),
    ],
    out_specs=[
        pl.BlockSpec((BLK, D), lambda i: (i, 0)),
        pl.BlockSpec((BLK, D), lambda i: (i, 0)),
        pl.BlockSpec((32, D), lambda i: (0, 0)),
    ],
    out_shape=[
        jax.ShapeDtypeStruct((N, D), jnp.float32),
        jax.ShapeDtypeStruct((N, D), jnp.float32),
        jax.ShapeDtypeStruct((32, D), jnp.float32),
    ],
)


_sc_mesh = plsc.VectorSubcoreMesh(core_axis_name="c", subcore_axis_name="s")


@functools.partial(
    pl.kernel,
    out_type=jax.ShapeDtypeStruct((NW, SLAB), jnp.float32),
    mesh=_sc_mesh,
    scratch_types=[
        pltpu.VMEM((SLAB,), jnp.float32),
        pltpu.VMEM((SLAB,), jnp.float32),
        pltpu.VMEM((32 * CPT,), jnp.float32),
        pltpu.VMEM((CHUNK,), jnp.int32),
        pltpu.VMEM((CHUNK,), jnp.int32),
        pltpu.VMEM((CHUNK,), jnp.int32),
        pltpu.VMEM((CHUNK,), jnp.int32),
        pltpu.SemaphoreType.DMA,
        pltpu.SemaphoreType.DMA,
        pltpu.SemaphoreType.DMA,
        pltpu.SemaphoreType.DMA,
    ],
    compiler_params=pltpu.CompilerParams(needs_layout_passes=False),
)
def _sc_edge_kernel(a_hbm, t_hbm, pw_hbm, q_hbm, p_hbm, out_hbm,
                    a_v, m_v, t_v, pw0, q0, pw1, q1, sp0, sq0, sp1, sq1):
    wid = lax.axis_index("s") * NC + lax.axis_index("c")
    pltpu.sync_copy(a_hbm.at[wid], a_v)
    pltpu.sync_copy(t_hbm.at[wid], t_v)

    neg = jnp.full((L,), -jnp.inf, dtype=jnp.float32)

    def init_body(i, carry):
        m_v[pl.ds(i * L, L)] = neg
        return carry

    lax.fori_loop(0, SLAB // L, init_body, 0)

    a_cols = [a_v.at[pl.ds(c * N, N)] for c in range(CPT)]
    m_cols = [m_v.at[pl.ds(c * N, N)] for c in range(CPT)]
    t_cols = [t_v.at[pl.ds(c * 32, 32)] for c in range(CPT)]

    def load_group(pw_v, q_v, g):
        w = pw_v[pl.ds(g * L, L)]
        q = q_v[pl.ds(g * L, L)]
        p = w & 0x3FFF
        cd = w >> 14
        vals = []
        for c in range(CPT):
            a_c = plsc.load_gather(a_cols[c], [p])
            t_c = plsc.load_gather(t_cols[c], [cd])
            vals.append(a_c + t_c)
        return q, vals

    def make_batch_body(pw_v, q_v):
        return functools.partial(batch_body, pw_v, q_v)

    def batch_body(pw_v, q_v, b, carry):
        base = b * BG
        viol = jnp.zeros((L,), dtype=jnp.bool_)
        for j in range(BG):
            q, vals = load_group(pw_v, q_v, base + j)
            needs = []
            for c in range(CPT):
                cur = plsc.load_gather(m_cols[c], [q])
                needs.append(vals[c] > cur)
            for c in range(CPT):
                plsc.store_scatter(m_cols[c], [q], vals[c], mask=needs[c])
            for c in range(CPT):
                cur2 = plsc.load_gather(m_cols[c], [q])
                viol = viol | (needs[c] & (vals[c] > cur2))

        @pl.when(jnp.any(viol))
        def _slow():
            for j in range(BG):
                q, vals = load_group(pw_v, q_v, base + j)
                needs0 = []
                for c in range(CPT):
                    cur0 = plsc.load_gather(m_cols[c], [q])
                    needs0.append(vals[c] > cur0)

                def cond(needs):
                    return jnp.any(needs[0] | needs[1] | needs[2] | needs[3])

                def body(needs):
                    for c in range(CPT):
                        plsc.store_scatter(m_cols[c], [q], vals[c],
                                           mask=needs[c])
                    new = []
                    for c in range(CPT):
                        cur = plsc.load_gather(m_cols[c], [q])
                        new.append(needs[c] & (vals[c] > cur))
                    return tuple(new)

                lax.while_loop(cond, body, tuple(needs0))
        return carry

    def start_copy(k, pwb, qb, sem_p, sem_q):
        kc = jnp.minimum(k, NCHUNK - 1) * CHUNK
        pltpu.async_copy(pw_hbm.at[pl.ds(kc, CHUNK)], pwb, sem_p)
        pltpu.async_copy(q_hbm.at[pl.ds(kc, CHUNK)], qb, sem_q)

    def wait_copy(pwb, qb, sem_p, sem_q):
        pltpu.make_async_copy(pw_hbm.at[pl.ds(0, CHUNK)], pwb, sem_p).wait()
        pltpu.make_async_copy(q_hbm.at[pl.ds(0, CHUNK)], qb, sem_q).wait()

    def process(pw_v, q_v):
        lax.fori_loop(0, GROUPS // BG, make_batch_body(pw_v, q_v), 0)

    start_copy(0, pw0, q0, sp0, sq0)

    def chunk_pair(kk, carry):
        k0 = kk * 2
        wait_copy(pw0, q0, sp0, sq0)
        start_copy(k0 + 1, pw1, q1, sp1, sq1)
        process(pw0, q0)
        wait_copy(pw1, q1, sp1, sq1)
        start_copy(k0 + 2, pw0, q0, sp0, sq0)
        process(pw1, q1)
        return carry

    lax.fori_loop(0, NCHUNK // 2, chunk_pair, 0)
    wait_copy(pw0, q0, sp0, sq0)

    pltpu.sync_copy(p_hbm.at[wid], a_v)

    def ep_body(i, carry):
        sl = pl.ds(i * L, L)
        m_v[sl] = jnp.maximum(m_v[sl] - a_v[sl], 0.0)
        return carry

    lax.fori_loop(0, SLAB // L, ep_body, 0)
    pltpu.sync_copy(m_v, out_hbm.at[wid])


import numpy as _np

_COMBOS = _np.array(
    [[i - 1, j - 1, k - 1]
     for i in range(3) for j in range(3) for k in range(3)] + [[0, 0, 0]] * 5,
    dtype=_np.float32)


def kernel(x, pos, fps_pos, batch, frac_pos, trans_vec, scale,
           q_inds, p_inds, offset, W1, b1):
    W1a = W1[:D]
    W1b = W1[D:]
    rowsum = (trans_vec[0] * scale[0]).sum(axis=1)
    W1c = rowsum[:, None] * W1b

    A, P, T = _prep_call(x, pos, jnp.asarray(_COMBOS), W1a, W1b, W1c,
                         b1.reshape(1, D))

    A_s = A.reshape(N, NW, CPT).transpose(1, 2, 0).reshape(NW, SLAB)
    P_s = P.reshape(N, NW, CPT).transpose(1, 2, 0).reshape(NW, SLAB)
    T_s = T.reshape(32, NW, CPT).transpose(1, 2, 0).reshape(NW, 32 * CPT)

    offi = offset.astype(jnp.int32) + 1
    code = offi[:, 0] * 9 + offi[:, 1] * 3 + offi[:, 2]
    pw = p_inds.astype(jnp.int32) | (code << 14)
    qi = q_inds.astype(jnp.int32)

    out_s = _sc_edge_kernel(A_s, T_s, pw, qi, P_s)
    out = out_s.reshape(NW, CPT, N).transpose(2, 0, 1).reshape(N, D)
    return (out, pos, fps_pos, batch, frac_pos, trans_vec, scale)
